# baseline (speedup 1.0000x reference)
import functools

import jax
import jax.numpy as jnp
from jax import lax
from jax.experimental import pallas as pl
from jax.experimental.pallas import tpu as pltpu
from jax.experimental.pallas import tpu_sc as plsc

_B = 4096
_L = 2048
_BB = 512


def _onehot_block(seq_ref, out_ref):
    s = seq_ref[...]
    for i in range(4):
        out_ref[:, i, :] = (s == i).astype(jnp.float32)


def _tc_kernel(seq):
    return pl.pallas_call(
        _onehot_block,
        grid=(_B // _BB,),
        in_specs=[pl.BlockSpec((_BB, _L), lambda i: (i, 0))],
        out_specs=pl.BlockSpec((_BB, 4, _L), lambda i: (i, 0, 0)),
        out_shape=jax.ShapeDtypeStruct((_B, 4, _L), jnp.float32),
    )(seq)


_NC = 2
_NS = 16
_NW = _NC * _NS
_RPW = _B // _NW
_RC = 4
_NCHUNK = _RPW // _RC
_NSLICE = _L // 16


@functools.partial(
    pl.kernel,
    mesh=plsc.VectorSubcoreMesh(core_axis_name="c", subcore_axis_name="s"),
    out_type=jax.ShapeDtypeStruct((_B, 4, _L), jnp.float32),
    scratch_types=[
        pltpu.VMEM((_RC, _L), jnp.int32),
        pltpu.VMEM((_RC, 4, _L), jnp.float32),
    ],
)
def _sc_onehot(seq_hbm, out_hbm, seq_v, out_v):
    wid = lax.axis_index("s") * _NC + lax.axis_index("c")
    base = wid * _RPW

    def chunk_body(c, carry):
        row0 = base + c * _RC
        pltpu.sync_copy(seq_hbm.at[pl.ds(row0, _RC)], seq_v)

        def slice_body(j, carry2):
            off = j * 16
            for r in range(_RC):
                s = seq_v[r, pl.ds(off, 16)]
                for i in range(4):
                    out_v[r, i, pl.ds(off, 16)] = jnp.where(
                        s == i, jnp.float32(1.0), jnp.float32(0.0)
                    )
            return carry2

        lax.fori_loop(0, _NSLICE, slice_body, 0, unroll=False)
        pltpu.sync_copy(out_v, out_hbm.at[pl.ds(row0, _RC)])
        return carry

    lax.fori_loop(0, _NCHUNK, chunk_body, 0, unroll=False)


def kernel(sequences):
    seq = sequences.astype(jnp.int32)
    return _sc_onehot(seq)
